# baseline (speedup 1.0000x reference)
import functools

import jax
import jax.numpy as jnp
from jax import lax
from jax.experimental import pallas as pl
from jax.experimental.pallas import tpu as pltpu
from jax.experimental.pallas import tpu_sc as plsc

B = 16384
DIM = 64
TEXT_DIM = 1536


def _sc_gather(P, model_id):
    info = plsc.get_sparse_core_info()
    nw = info.num_cores * info.num_subcores
    b_per_w = B // nw
    mesh = plsc.VectorSubcoreMesh(core_axis_name="c", subcore_axis_name="s")

    @functools.partial(
        pl.kernel,
        mesh=mesh,
        out_type=jax.ShapeDtypeStruct((B, DIM), jnp.float32),
        scratch_types=[
            pltpu.VMEM((b_per_w,), jnp.int32),
            pltpu.VMEM((b_per_w, DIM), jnp.float32),
            pltpu.SemaphoreType.DMA,
        ],
        compiler_params=pltpu.CompilerParams(use_tc_tiling_on_sc=False),
    )
    def gather_kernel(table_hbm, idx_hbm, out_hbm, idx_v, rows_v, sem):
        wid = lax.axis_index("s") * info.num_cores + lax.axis_index("c")
        base = wid * b_per_w
        pltpu.sync_copy(idx_hbm.at[pl.ds(base, b_per_w)], idx_v)
        pltpu.async_copy(table_hbm.at[idx_v], rows_v, sem).wait()
        pltpu.sync_copy(rows_v, out_hbm.at[pl.ds(base, b_per_w)])

    return gather_kernel(P, model_id)


def _tc_body(x_ref, e_ref, w_ref, wc_ref, o_ref):
    pe = lax.dot_general(
        x_ref[...], w_ref[...],
        (((1,), (1,)), ((), ())),
        preferred_element_type=jnp.float32,
    )
    e = e_ref[...]
    s = jnp.sum(e * pe * wc_ref[...], axis=1)
    n = jnp.sum(e * e, axis=1)
    o_ref[...] = s / jnp.maximum(jnp.sqrt(n), 1e-12)


def _tc_combine(prompt, emb, W_proj, W_cls, block_b=2048):
    grid = (B // block_b,)
    return pl.pallas_call(
        _tc_body,
        grid=grid,
        in_specs=[
            pl.BlockSpec((block_b, TEXT_DIM), lambda i: (i, 0)),
            pl.BlockSpec((block_b, DIM), lambda i: (i, 0)),
            pl.BlockSpec((DIM, TEXT_DIM), lambda i: (0, 0)),
            pl.BlockSpec((1, DIM), lambda i: (0, 0)),
        ],
        out_specs=pl.BlockSpec((block_b,), lambda i: (i,)),
        out_shape=jax.ShapeDtypeStruct((B,), jnp.float32),
    )(prompt, emb, W_proj, W_cls)


def kernel(model_id, prompt, P, W_proj, W_cls):
    emb = _sc_gather(P, model_id)
    return _tc_combine(prompt, emb, W_proj, W_cls)


# speedup vs baseline: 1.6570x; 1.6570x over previous
import functools

import jax
import jax.numpy as jnp
from jax import lax
from jax.experimental import pallas as pl
from jax.experimental.pallas import tpu as pltpu
from jax.experimental.pallas import tpu_sc as plsc

B = 16384
DIM = 64
TEXT_DIM = 1536


def _sc_gather(P, model_id):
    info = plsc.get_sparse_core_info()
    nw = info.num_cores * info.num_subcores
    b_per_w = B // nw
    mesh = plsc.VectorSubcoreMesh(core_axis_name="c", subcore_axis_name="s")

    @functools.partial(
        pl.kernel,
        mesh=mesh,
        out_type=jax.ShapeDtypeStruct((B, DIM), jnp.float32),
        scratch_types=[
            pltpu.VMEM((b_per_w,), jnp.int32),
            pltpu.VMEM((b_per_w, DIM), jnp.float32),
            pltpu.SemaphoreType.DMA,
            pltpu.SemaphoreType.DMA,
        ],
    )
    def gather_kernel(table_hbm, idx_hbm, out_hbm, idx_s, rows_v, sem_i, sem):
        wid = lax.axis_index("s") * info.num_cores + lax.axis_index("c")
        base = wid * b_per_w
        pltpu.async_copy(idx_hbm.at[pl.ds(base, b_per_w)], idx_s, sem_i).wait()

        def fire(blk, _):
            r0 = blk * 16
            v = idx_s[pl.ds(r0, 16)]
            for k in range(16):
                pltpu.make_async_copy(
                    table_hbm.at[v[k]], rows_v.at[r0 + k], sem
                ).start()
            return _

        lax.fori_loop(0, b_per_w // 16, fire, 0)
        pltpu.make_async_copy(
            table_hbm.at[pl.ds(0, b_per_w)], rows_v, sem
        ).wait()
        pltpu.sync_copy(rows_v, out_hbm.at[pl.ds(base, b_per_w)])

    return gather_kernel(P, model_id)


def _tc_body(x_ref, e_ref, w_ref, wc_ref, o_ref):
    pe = lax.dot_general(
        x_ref[...], w_ref[...],
        (((1,), (1,)), ((), ())),
        preferred_element_type=jnp.float32,
    )
    e = e_ref[...]
    s = jnp.sum(e * pe * wc_ref[...], axis=1)
    n = jnp.sum(e * e, axis=1)
    o_ref[...] = s / jnp.maximum(jnp.sqrt(n), 1e-12)


def _tc_combine(prompt, emb, W_proj, W_cls, block_b=2048):
    grid = (B // block_b,)
    return pl.pallas_call(
        _tc_body,
        grid=grid,
        in_specs=[
            pl.BlockSpec((block_b, TEXT_DIM), lambda i: (i, 0)),
            pl.BlockSpec((block_b, DIM), lambda i: (i, 0)),
            pl.BlockSpec((DIM, TEXT_DIM), lambda i: (0, 0)),
            pl.BlockSpec((1, DIM), lambda i: (0, 0)),
        ],
        out_specs=pl.BlockSpec((block_b,), lambda i: (i,)),
        out_shape=jax.ShapeDtypeStruct((B,), jnp.float32),
    )(prompt, emb, W_proj, W_cls)


def kernel(model_id, prompt, P, W_proj, W_cls):
    emb = _sc_gather(P, model_id)
    return _tc_combine(prompt, emb, W_proj, W_cls)


# speedup vs baseline: 2.6531x; 1.6012x over previous
import functools

import jax
import jax.numpy as jnp
from jax import lax
from jax.experimental import pallas as pl
from jax.experimental.pallas import tpu as pltpu
from jax.experimental.pallas import tpu_sc as plsc

B = 16384
DIM = 64
NUM_MODELS = 1000000
TEXT_DIM = 1536

_LAST_TILE = (NUM_MODELS - 1) // 128
_TAIL = _LAST_TILE * 128
_MAIN_LAST = _LAST_TILE - 1


def _sc_gather_t(Pt, model_id):
    info = plsc.get_sparse_core_info()
    nw = info.num_cores * info.num_subcores
    b_per_w = B // nw
    nblk = b_per_w // 16
    R = 8
    mesh = plsc.VectorSubcoreMesh(core_axis_name="c", subcore_axis_name="s")

    @functools.partial(
        pl.kernel,
        mesh=mesh,
        out_type=jax.ShapeDtypeStruct((B, DIM), jnp.float32),
        scratch_types=[
            pltpu.VMEM((b_per_w,), jnp.int32),
            pltpu.VMEM((R, DIM, 128), jnp.float32),
            pltpu.VMEM((DIM, 64), jnp.float32),
            pltpu.VMEM((R, 1, DIM), jnp.float32),
            pltpu.SemaphoreType.DMA,
            pltpu.SemaphoreType.DMA,
            [pltpu.SemaphoreType.DMA] * R,
            [pltpu.SemaphoreType.DMA] * R,
        ],
        compiler_params=pltpu.CompilerParams(needs_layout_passes=False),
    )
    def gather_kernel(table_hbm, idx_hbm, out_hbm, idx_s, win_v, tail_v,
                      row_v, sem_i, sem_t, wsems, osems):
        wid = lax.axis_index("s") * info.num_cores + lax.axis_index("c")
        base = wid * b_per_w
        pltpu.async_copy(idx_hbm.at[pl.ds(base, b_per_w)], idx_s, sem_i).wait()
        pltpu.async_copy(
            table_hbm.at[:, pl.ds(_TAIL, 64)], tail_v, sem_t
        ).wait()

        def win_start(idv, slot):
            tc = jnp.minimum(lax.shift_right_logical(idv, 7), _MAIN_LAST)
            pltpu.make_async_copy(
                table_hbm.at[:, pl.ds(tc * 128, 128)],
                win_v.at[slot],
                wsems[slot],
            ).start()

        def win_wait(slot):
            pltpu.make_async_copy(
                table_hbm.at[:, pl.ds(0, 128)], win_v.at[slot], wsems[slot]
            ).wait()

        def out_start(slot, row):
            pltpu.make_async_copy(
                row_v.at[slot], out_hbm.at[pl.ds(base + row, 1)], osems[slot]
            ).start()

        def out_wait(slot):
            pltpu.make_async_copy(
                row_v.at[slot], out_hbm.at[pl.ds(0, 1)], osems[slot]
            ).wait()

        def extract(idv, slot):
            tc = jnp.minimum(lax.shift_right_logical(idv, 7), _MAIN_LAST)
            l_main = jnp.minimum(idv - tc * 128, 127)
            l_tail = jnp.maximum(jnp.minimum(idv - _TAIL, 63), 0)
            is_tail = (idv >= _TAIL).astype(jnp.int32)
            lm = jnp.full((16,), l_main, jnp.int32)
            lt = jnp.full((16,), l_tail, jnp.int32)
            sel = jnp.full((16,), is_tail, jnp.int32) > 0
            for a in range(4):
                c = lax.iota(jnp.int32, 16) + (16 * a)
                gm = plsc.load_gather(win_v.at[slot], [c, lm])
                gt = plsc.load_gather(tail_v, [c, lt])
                row_v[slot, 0, pl.ds(16 * a, 16)] = jnp.where(sel, gt, gm)

        def item(k, r0, v, va8, first, last):
            slot = k % R
            win_wait(slot)
            if not (first and k < R):
                out_wait(slot)
            extract(v[k], slot)
            out_start(slot, r0 + k)
            if not last:
                nid = v[k + R] if k < 16 - R else va8[k + R - 8]
                win_start(nid, slot)
            elif k < 16 - R:
                win_start(v[k + R], slot)

        v0 = idx_s[pl.ds(0, 16)]
        for p in range(R):
            win_start(v0[p], p)

        va8_0 = idx_s[pl.ds(8, 16)]
        for k in range(16):
            item(k, 0, v0, va8_0, first=True, last=False)

        def blk_body(blk, _):
            r0 = blk * 16
            v = idx_s[pl.ds(r0, 16)]
            va8 = idx_s[pl.ds(r0 + 8, 16)]
            for k in range(16):
                item(k, r0, v, va8, first=False, last=False)
            return _

        lax.fori_loop(1, nblk - 1, blk_body, 0)

        r0 = (nblk - 1) * 16
        v = idx_s[pl.ds(r0, 16)]
        for k in range(16):
            item(k, r0, v, None, first=False, last=True)
        for s in range(R):
            out_wait(s)

    return gather_kernel(Pt, model_id)


def _mm_body(x_ref, w_ref, wc_ref, o_ref):
    pe = lax.dot_general(
        x_ref[...], w_ref[...],
        (((1,), (1,)), ((), ())),
        preferred_element_type=jnp.float32,
    )
    o_ref[...] = pe * wc_ref[...]


def _tc_matmul(prompt, W_proj, W_cls, block_b=2048):
    grid = (B // block_b,)
    return pl.pallas_call(
        _mm_body,
        grid=grid,
        in_specs=[
            pl.BlockSpec((block_b, TEXT_DIM), lambda i: (i, 0)),
            pl.BlockSpec((DIM, TEXT_DIM), lambda i: (0, 0)),
            pl.BlockSpec((1, DIM), lambda i: (0, 0)),
        ],
        out_specs=pl.BlockSpec((block_b, DIM), lambda i: (i, 0)),
        out_shape=jax.ShapeDtypeStruct((B, DIM), jnp.float32),
    )(prompt, W_proj, W_cls)


def _cb_body(e_ref, q_ref, o_ref):
    e = e_ref[...]
    s = jnp.sum(e * q_ref[...], axis=1)
    n = jnp.sum(e * e, axis=1)
    o_ref[...] = s / jnp.maximum(jnp.sqrt(n), 1e-12)


def _tc_combine(emb, q, block_b=8192):
    grid = (B // block_b,)
    return pl.pallas_call(
        _cb_body,
        grid=grid,
        in_specs=[
            pl.BlockSpec((block_b, DIM), lambda i: (i, 0)),
            pl.BlockSpec((block_b, DIM), lambda i: (i, 0)),
        ],
        out_specs=pl.BlockSpec((block_b,), lambda i: (i,)),
        out_shape=jax.ShapeDtypeStruct((B,), jnp.float32),
    )(emb, q)


def kernel(model_id, prompt, P, W_proj, W_cls):
    Pt = P.T
    emb = _sc_gather_t(Pt, model_id)
    q = _tc_matmul(prompt, W_proj, W_cls)
    return _tc_combine(emb, q)
